# baseline (speedup 1.0000x reference)
import jax
import jax.numpy as jnp
from jax import lax
from jax.experimental import pallas as pl
from jax.experimental.pallas import tpu as pltpu
from jax.experimental.pallas import tpu_sc as plsc

MIN_NORM = 1e-15
BALL_EPS = 0.004
MIN_CURVATURE = 1e-06
LN2 = 0.6931471805599453

N_SUBCORES = 16
N_CORES = 2
E_PER_TILE = 10080
ACC_ROWS = 10008
S1W = 272
S2W = 64
C1 = 24
C2 = 80


def _expmap0_tc(u, c):
    sqrt_c = jnp.sqrt(c)
    u_norm = jnp.maximum(
        jnp.sqrt(jnp.sum(u * u, axis=-1, keepdims=True)), MIN_NORM)
    g = jnp.tanh(jnp.clip(sqrt_c * u_norm, -15.0, 15.0)) * u / (sqrt_c * u_norm)
    maxnorm = (1.0 - BALL_EPS) / sqrt_c
    g_norm = jnp.maximum(
        jnp.sqrt(jnp.sum(g * g, axis=-1, keepdims=True)), MIN_NORM)
    return jnp.where(g_norm > maxnorm, g / g_norm * maxnorm, g)


def _stable_body(h_ref, p_ref, hq_ref, ws_ref, wpp_ref, wq_ref, wqb_ref, c_ref,
                 s1_ref, s2_ref, q_ref):
    c = c_ref[0]
    h = h_ref[...]
    a = jnp.dot(h, ws_ref[...], preferred_element_type=jnp.float32)
    hh = _expmap0_tc(h, c)
    x2 = jnp.sum(hh * hh, axis=-1, keepdims=True)
    p1 = jnp.dot(p_ref[...], wpp_ref[...], preferred_element_type=jnp.float32)
    pad = jnp.zeros((h.shape[0], S1W - 257), jnp.float32)
    s1_ref[...] = jnp.concatenate([a, hh, x2, pad], axis=1)
    s2_ref[...] = p1
    q_ref[...] = (jnp.dot(hq_ref[...], wq_ref[...],
                          preferred_element_type=jnp.float32) + wqb_ref[...])


def _rtable_body(r_ref, wr_ref, wpr_ref, c_ref, r1_ref, r2_ref):
    c = c_ref[0]
    r = r_ref[...]
    b = jnp.dot(r, wr_ref[...], preferred_element_type=jnp.float32)
    rh = _expmap0_tc(r, c)
    y2 = jnp.sum(rh * rh, axis=-1, keepdims=True)
    p2 = jnp.dot(r, wpr_ref[...], preferred_element_type=jnp.float32)
    pad = jnp.zeros((r.shape[0], S1W - 257), jnp.float32)
    r1_ref[...] = jnp.concatenate([b, rh, y2, pad], axis=1)
    r2_ref[...] = p2


def _final_body(pm_ref, pp_ref, wh_ref, c_ref, hid_ref, path_ref):
    c = c_ref[0]
    sqrt_c = jnp.sqrt(c)
    m = pm_ref[0] + pm_ref[1]
    a = jnp.dot(m, wh_ref[...], preferred_element_type=jnp.float32)
    a_norm = jnp.maximum(
        jnp.sqrt(jnp.sum(a * a, axis=-1, keepdims=True)), MIN_NORM)
    g = jnp.tanh(jnp.clip(sqrt_c * a_norm, -15.0, 15.0)) * a / (sqrt_c * a_norm)
    maxnorm = (1.0 - BALL_EPS) / sqrt_c
    g_norm = jnp.maximum(
        jnp.sqrt(jnp.sum(g * g, axis=-1, keepdims=True)), MIN_NORM)
    y = jnp.where(g_norm > maxnorm, g / g_norm * maxnorm, g)
    y_norm = jnp.maximum(
        jnp.sqrt(jnp.sum(y * y, axis=-1, keepdims=True)), MIN_NORM)
    z = jnp.clip(sqrt_c * y_norm, -1.0 + 1e-05, 1.0 - 1e-05)
    art = 0.5 * (jnp.log1p(z) - jnp.log1p(-z))
    hid_ref[...] = y / y_norm / sqrt_c * art
    path_ref[...] = pp_ref[0] + pp_ref[1]


def _lane_sum(v):
    iota = lax.broadcasted_iota(jnp.int32, (16,), 0)
    for k in (8, 4, 2, 1):
        v = v + v.at[jnp.bitwise_xor(iota, k)].get(
            mode="promise_in_bounds")
    return v


def _lane0(v):
    return v.at[jnp.zeros((16,), jnp.int32)].get(mode="promise_in_bounds")


def _attention_alpha(sbuf, rbuf, qbuf, i, wv, wb_vec):
    aacc = None
    for j in range(8):
        av = sbuf[i, pl.ds(j * 16, 16)]
        bv = rbuf[i, pl.ds(j * 16, 16)]
        qv = qbuf[i, pl.ds(j * 16, 16)]
        t = jnp.maximum(av + bv + qv, 0.0) * wv[j]
        aacc = t if aacc is None else aacc + t
    lo_v = jnp.full((16,), jnp.sum(aacc), jnp.float32) + wb_vec
    return 1.0 / (1.0 + jnp.exp(-lo_v))


ZCH = 24


def _zero_and_barrier(stage, acc, sid, stage_rows, width):
    zeros16 = jnp.zeros((16,), jnp.float32)

    def zrow(i, carry):
        for j in range(width // 16):
            stage[i, pl.ds(j * 16, 16)] = zeros16
        return carry
    lax.fori_loop(0, min(stage_rows, ZCH), zrow, 0)
    src = stage if stage_rows == ZCH else stage.at[pl.ds(0, ZCH)]
    n_zchunk = ACC_ROWS // ZCH

    def zacc(k, carry):
        cidx = sid + k * N_SUBCORES

        @pl.when(cidx < n_zchunk)
        def _():
            off = pl.multiple_of(cidx * ZCH, 8)
            pltpu.sync_copy(src, acc.at[pl.ds(off, ZCH)])
        return carry
    lax.fori_loop(0, (n_zchunk + N_SUBCORES - 1) // N_SUBCORES, zacc, 0)
    plsc.subcore_barrier()


def _writeout(acc, out_hbm, cid, sid, n_rows, chunk):
    n_zchunk = n_rows // chunk

    def wout(k, carry):
        cidx = sid + k * N_SUBCORES

        @pl.when(cidx < n_zchunk)
        def _():
            off = pl.multiple_of(cidx * chunk, 8)
            pltpu.sync_copy(acc.at[pl.ds(off, chunk)],
                            out_hbm.at[cid, pl.ds(off, chunk)])
        return carry
    lax.fori_loop(0, (n_zchunk + N_SUBCORES - 1) // N_SUBCORES, wout, 0)


def _msg_body(s_hbm, r_hbm, q_hbm, sub_hbm, rel_hbm, ridx_hbm, obj_hbm,
              w_hbm, cst_hbm, out_hbm, aout_hbm,
              sbuf0, rbuf0, qbuf0, sbuf1, rbuf1, qbuf1, stage, abuf,
              isub0, irel0, iridx0, iobj0, isub1, irel1, iridx1, iobj1,
              wbuf, cbuf, acc, semi0, semi1, semg0, semg1):
    cid = lax.axis_index("c")
    sid = lax.axis_index("s")
    tid = cid * N_SUBCORES + sid

    pltpu.sync_copy(w_hbm, wbuf)
    pltpu.sync_copy(cst_hbm, cbuf)
    wv = [wbuf[j, :] for j in range(8)]
    c_vec = cbuf[0, :]
    sqrtc_vec = cbuf[1, :]
    invsqrtc_vec = cbuf[2, :]
    maxnorm_vec = cbuf[3, :]
    wb_vec = cbuf[4, :]
    c2_vec = cbuf[5, :]
    one_i = jnp.full((16,), 1, jnp.int32)
    k23_i = jnp.full((16,), 23, jnp.int32)
    lane0 = lax.broadcasted_iota(jnp.int32, (16,), 0) == 0

    _zero_and_barrier(stage, acc, sid, C1, 128)

    idxs = [(isub0, irel0, iridx0, iobj0), (isub1, irel1, iridx1, iobj1)]
    bufs = [(sbuf0, rbuf0, qbuf0), (sbuf1, rbuf1, qbuf1)]
    semi = [semi0, semi1]
    semg = [semg0, semg1]

    def fire_idx(t, b):
        base = pl.multiple_of(tid * E_PER_TILE + t * C1, 8)
        for src, dst in zip((sub_hbm, rel_hbm, ridx_hbm, obj_hbm), idxs[b]):
            pltpu.async_copy(src.at[pl.ds(base, C1)], dst, semi[b])

    def wait_idx(b):
        for src, dst in zip((sub_hbm, rel_hbm, ridx_hbm, obj_hbm), idxs[b]):
            pltpu.make_async_copy(src.at[pl.ds(0, C1)], dst, semi[b]).wait()

    def fire_gathers(b):
        isub, irel, iridx, _ = idxs[b]
        sb, rb, qb = bufs[b]
        pltpu.async_copy(s_hbm.at[isub], sb, semg[b])
        pltpu.async_copy(r_hbm.at[irel], rb, semg[b])
        pltpu.async_copy(q_hbm.at[iridx], qb, semg[b])

    def wait_gathers(b):
        isub, irel, iridx, _ = idxs[b]
        sb, rb, qb = bufs[b]
        pltpu.make_async_copy(s_hbm.at[isub], sb, semg[b]).wait()
        pltpu.make_async_copy(r_hbm.at[irel], rb, semg[b]).wait()
        pltpu.make_async_copy(q_hbm.at[iridx], qb, semg[b]).wait()

    def do_chunk(t, b):
        sb, rb, qb = bufs[b]
        iobj = idxs[b][3]

        @plsc.parallel_loop(0, C1, unroll=8)
        def edge_one(i):
            alpha = _attention_alpha(sb, rb, qb, i, wv, wb_vec)
            plsc.store_scatter(abuf, [jnp.full((16,), i, jnp.int32)], alpha,
                               mask=lane0)
            xs = []
            ys = []
            xyacc = None
            for j in range(8):
                xv = sb[i, pl.ds(128 + j * 16, 16)]
                yv = rb[i, pl.ds(128 + j * 16, 16)]
                xs.append(xv)
                ys.append(yv)
                pv = xv * yv
                xyacc = pv if xyacc is None else xyacc + pv
            xy = jnp.full((16,), jnp.sum(xyacc), jnp.float32)
            x2 = _lane0(sb[i, pl.ds(256, 16)])
            y2 = _lane0(rb[i, pl.ds(256, 16)])

            two_cxy = 2.0 * (c_vec * xy)
            den = jnp.maximum(1.0 + two_cxy + c2_vec * (x2 * y2), MIN_NORM)
            u = (1.0 + two_cxy + c_vec * y2) / den
            v = (1.0 - c_vec * x2) / den
            n2 = u * u * x2 + 2.0 * (u * v * xy) + v * v * y2
            n2g = jnp.maximum(n2, 1e-30)
            bits = plsc.bitcast(n2g, jnp.int32)
            k0 = (jnp.full((16,), 0x5F3759DF, jnp.int32) -
                  lax.shift_right_logical(bits, one_i))
            yr = plsc.bitcast(k0, jnp.float32)
            half = 0.5 * n2g
            for _ in range(3):
                yr = yr * (1.5 - half * yr * yr)
            n = n2g * yr
            norm = jnp.maximum(n, MIN_NORM)
            r = jnp.minimum(1.0, maxnorm_vec / norm)
            y_norm = jnp.maximum(r * n, MIN_NORM)
            z = jnp.clip(sqrtc_vec * y_norm, -1.0 + 1e-05, 1.0 - 1e-05)
            qr = (1.0 + z) / (1.0 - z)
            qb_ = plsc.bitcast(qr, jnp.int32)
            e_i = lax.shift_right_logical(qb_, k23_i) - 127
            m_b = jnp.bitwise_or(jnp.bitwise_and(qb_, 0x007FFFFF), 0x3F800000)
            m = plsc.bitcast(m_b, jnp.float32)
            t_ = (m - 1.0) / (m + 1.0)
            t2 = t_ * t_
            lnm = t_ * (2.0 + t2 * (0.6666666666 + t2 * (0.4 + t2 * (
                0.2857142857 + t2 * 0.2222222222))))
            lnq = e_i.astype(jnp.float32) * LN2 + lnm
            art = 0.5 * lnq
            tt = (r / y_norm) * invsqrtc_vec * art
            o1 = alpha * (tt * u)
            o2 = alpha * (tt * v)
            for j in range(8):
                stage[i, pl.ds(j * 16, 16)] = o1 * xs[j] + o2 * ys[j]

        pltpu.sync_copy(stage, acc.at[iobj], add=True)
        base = pl.multiple_of(tid * E_PER_TILE + t * C1, 8)
        pltpu.sync_copy(abuf, aout_hbm.at[pl.ds(base, C1)])

    nch = E_PER_TILE // C1
    fire_idx(jnp.int32(0), 0)
    wait_idx(0)
    fire_gathers(0)
    fire_idx(jnp.int32(1), 1)

    def pair(t2, carry):
        t = t2 * 2
        wait_gathers(0)
        wait_idx(1)
        fire_gathers(1)
        do_chunk(t, 0)

        @pl.when(t + 2 < nch)
        def _():
            fire_idx(t + 2, 0)
        wait_gathers(1)

        @pl.when(t + 2 < nch)
        def _():
            wait_idx(0)
            fire_gathers(0)
        do_chunk(t + 1, 1)

        @pl.when(t + 2 < nch)
        def _():
            fire_idx(t + 3, 1)
        return carry

    lax.fori_loop(0, nch // 2, pair, 0)
    plsc.subcore_barrier()
    _writeout(acc, out_hbm, cid, sid, 10000, 40)


def _path_body(s_hbm, r_hbm, a_hbm, sub_hbm, rel_hbm, obj_hbm,
               out_hbm,
               sbuf0, rbuf0, abuf0, sbuf1, rbuf1, abuf1, stage,
               isub0, irel0, iobj0, isub1, irel1, iobj1,
               acc, semi0, semi1, semg0, semg1):
    cid = lax.axis_index("c")
    sid = lax.axis_index("s")
    tid = cid * N_SUBCORES + sid

    _zero_and_barrier(stage, acc, sid, C2, 64)

    idxs = [(isub0, irel0, iobj0), (isub1, irel1, iobj1)]
    bufs = [(sbuf0, rbuf0, abuf0), (sbuf1, rbuf1, abuf1)]
    semi = [semi0, semi1]
    semg = [semg0, semg1]

    def fire_idx(t, b):
        base = pl.multiple_of(tid * E_PER_TILE + t * C2, 8)
        for src, dst in zip((sub_hbm, rel_hbm, obj_hbm), idxs[b]):
            pltpu.async_copy(src.at[pl.ds(base, C2)], dst, semi[b])
        pltpu.async_copy(a_hbm.at[pl.ds(base, C2)], bufs[b][2], semi[b])

    def wait_idx(b):
        for src, dst in zip((sub_hbm, rel_hbm, obj_hbm), idxs[b]):
            pltpu.make_async_copy(src.at[pl.ds(0, C2)], dst, semi[b]).wait()
        pltpu.make_async_copy(a_hbm.at[pl.ds(0, C2)], bufs[b][2],
                              semi[b]).wait()

    def fire_gathers(b):
        isub, irel, _ = idxs[b]
        sb, rb, _ = bufs[b]
        pltpu.async_copy(s_hbm.at[isub], sb, semg[b])
        pltpu.async_copy(r_hbm.at[irel], rb, semg[b])

    def wait_gathers(b):
        isub, irel, _ = idxs[b]
        sb, rb, _ = bufs[b]
        pltpu.make_async_copy(s_hbm.at[isub], sb, semg[b]).wait()
        pltpu.make_async_copy(r_hbm.at[irel], rb, semg[b]).wait()

    def do_chunk(t, b):
        sb, rb, ab = bufs[b]
        iobj = idxs[b][2]

        @plsc.parallel_loop(0, C2, unroll=8)
        def edge_one(i):
            alpha = plsc.load_gather(ab, [jnp.full((16,), i, jnp.int32)])
            for j in range(4):
                pp = (sb[i, pl.ds(j * 16, 16)] +
                      rb[i, pl.ds(j * 16, 16)])
                pc = jnp.clip(pp, -15.0, 15.0)
                e2 = jnp.exp(2.0 * pc)
                th = (e2 - 1.0) / (e2 + 1.0)
                stage[i, pl.ds(j * 16, 16)] = alpha * th

        pltpu.sync_copy(stage, acc.at[iobj], add=True)

    nch = E_PER_TILE // C2
    fire_idx(jnp.int32(0), 0)
    wait_idx(0)
    fire_gathers(0)
    fire_idx(jnp.int32(1), 1)

    def pair(t2, carry):
        t = t2 * 2
        wait_gathers(0)
        wait_idx(1)
        fire_gathers(1)
        do_chunk(t, 0)

        @pl.when(t + 2 < nch)
        def _():
            fire_idx(t + 2, 0)
        wait_gathers(1)

        @pl.when(t + 2 < nch)
        def _():
            wait_idx(0)
            fire_gathers(0)
        do_chunk(t + 1, 1)

        @pl.when(t + 2 < nch)
        def _():
            fire_idx(t + 3, 1)
        return carry

    lax.fori_loop(0, nch // 2, pair, 0)
    plsc.subcore_barrier()
    _writeout(acc, out_hbm, cid, sid, 10000, 80)


def kernel(q_sub, q_rel, hidden, path_state, edges, nodes, old_nodes_new_idx,
           batchsize, rela_embed, Ws_attn, Wr_attn, Wqr_attn_w, Wqr_attn_b,
           w_alpha_w, w_alpha_b, W_h, W_path_prev, W_path_rel, curvature):
    c = jnp.maximum(curvature, MIN_CURVATURE)
    n_node = hidden.shape[0]
    vocab = rela_embed.shape[0]

    n_edge = edges.shape[0]
    n_tiles = N_CORES * N_SUBCORES
    epad = E_PER_TILE - n_edge // n_tiles

    def _pad_col(col, fill):
        col = jnp.asarray(col, jnp.int32).reshape(n_tiles, -1)
        col = jnp.pad(col, ((0, 0), (0, epad)), constant_values=fill)
        return col.reshape(-1)

    sub = _pad_col(edges[:, 4], 0)
    rel = _pad_col(edges[:, 2], 0)
    obj = _pad_col(edges[:, 5], n_node)
    r_idx = _pad_col(edges[:, 0], 0)
    vpad = (-vocab) % 16
    rela_p = jnp.pad(rela_embed, ((0, vpad), (0, 0)))
    vp = vocab + vpad
    hq = jnp.take(rela_embed, q_rel, axis=0)
    c_arr = jnp.reshape(c, (1,))
    sqrt_c = jnp.sqrt(c)
    consts = jnp.stack([c, sqrt_c, 1.0 / sqrt_c, (1.0 - BALL_EPS) / sqrt_c,
                        w_alpha_b[0], c * c, jnp.float32(0), jnp.float32(0)])
    consts = jnp.tile(consts[:, None], (1, 16))
    w_r = jnp.reshape(w_alpha_w, (8, 16))
    wqb = jnp.reshape(Wqr_attn_b, (1, 128))

    blk = n_node // 10
    s1_tab, s2_tab, q_tab = pl.pallas_call(
        _stable_body,
        out_shape=(
            jax.ShapeDtypeStruct((n_node, S1W), jnp.float32),
            jax.ShapeDtypeStruct((n_node, S2W), jnp.float32),
            jax.ShapeDtypeStruct((n_node, 128), jnp.float32),
        ),
        grid=(10,),
        in_specs=[
            pl.BlockSpec((blk, 128), lambda i: (i, 0)),
            pl.BlockSpec((blk, 64), lambda i: (i, 0)),
            pl.BlockSpec((blk, 128), lambda i: (i, 0)),
            pl.BlockSpec((128, 128), lambda i: (0, 0)),
            pl.BlockSpec((64, 64), lambda i: (0, 0)),
            pl.BlockSpec((128, 128), lambda i: (0, 0)),
            pl.BlockSpec((1, 128), lambda i: (0, 0)),
            pl.BlockSpec(memory_space=pltpu.SMEM),
        ],
        out_specs=(
            pl.BlockSpec((blk, S1W), lambda i: (i, 0)),
            pl.BlockSpec((blk, S2W), lambda i: (i, 0)),
            pl.BlockSpec((blk, 128), lambda i: (i, 0)),
        ),
    )(hidden, path_state, hq, Ws_attn, W_path_prev, Wqr_attn_w, wqb, c_arr)

    rblk = vp // 2
    r1_tab, r2_tab = pl.pallas_call(
        _rtable_body,
        out_shape=(
            jax.ShapeDtypeStruct((vp, S1W), jnp.float32),
            jax.ShapeDtypeStruct((vp, S2W), jnp.float32),
        ),
        grid=(2,),
        in_specs=[
            pl.BlockSpec((rblk, 128), lambda i: (i, 0)),
            pl.BlockSpec((128, 128), lambda i: (0, 0)),
            pl.BlockSpec((128, 64), lambda i: (0, 0)),
            pl.BlockSpec(memory_space=pltpu.SMEM),
        ],
        out_specs=(
            pl.BlockSpec((rblk, S1W), lambda i: (i, 0)),
            pl.BlockSpec((rblk, S2W), lambda i: (i, 0)),
        ),
    )(rela_p, Wr_attn, W_path_rel, c_arr)

    mesh = plsc.VectorSubcoreMesh(core_axis_name="c", subcore_axis_name="s")
    sc_params = pltpu.CompilerParams(
        needs_layout_passes=False, use_tc_tiling_on_sc=False)

    msg_part, alpha_e = pl.kernel(
        _msg_body,
        out_type=(
            jax.ShapeDtypeStruct((N_CORES, n_node, 128), jnp.float32),
            jax.ShapeDtypeStruct((n_tiles * E_PER_TILE,), jnp.float32),
        ),
        mesh=mesh,
        compiler_params=sc_params,
        scratch_types=[
            pltpu.VMEM((C1, S1W), jnp.float32),
            pltpu.VMEM((C1, S1W), jnp.float32),
            pltpu.VMEM((C1, 128), jnp.float32),
            pltpu.VMEM((C1, S1W), jnp.float32),
            pltpu.VMEM((C1, S1W), jnp.float32),
            pltpu.VMEM((C1, 128), jnp.float32),
            pltpu.VMEM((C1, 128), jnp.float32),
            pltpu.VMEM((C1,), jnp.float32),
            pltpu.VMEM((C1,), jnp.int32),
            pltpu.VMEM((C1,), jnp.int32),
            pltpu.VMEM((C1,), jnp.int32),
            pltpu.VMEM((C1,), jnp.int32),
            pltpu.VMEM((C1,), jnp.int32),
            pltpu.VMEM((C1,), jnp.int32),
            pltpu.VMEM((C1,), jnp.int32),
            pltpu.VMEM((C1,), jnp.int32),
            pltpu.VMEM((8, 16), jnp.float32),
            pltpu.VMEM((8, 16), jnp.float32),
            pltpu.VMEM_SHARED((ACC_ROWS, 128), jnp.float32),
            pltpu.SemaphoreType.DMA,
            pltpu.SemaphoreType.DMA,
            pltpu.SemaphoreType.DMA,
            pltpu.SemaphoreType.DMA,
        ],
    )(s1_tab, r1_tab, q_tab, sub, rel, r_idx, obj, w_r, consts)

    path_part = pl.kernel(
        _path_body,
        out_type=jax.ShapeDtypeStruct((N_CORES, n_node, 64), jnp.float32),
        mesh=mesh,
        compiler_params=sc_params,
        scratch_types=[
            pltpu.VMEM((C2, S2W), jnp.float32),
            pltpu.VMEM((C2, S2W), jnp.float32),
            pltpu.VMEM((C2,), jnp.float32),
            pltpu.VMEM((C2, S2W), jnp.float32),
            pltpu.VMEM((C2, S2W), jnp.float32),
            pltpu.VMEM((C2,), jnp.float32),
            pltpu.VMEM((C2, 64), jnp.float32),
            pltpu.VMEM((C2,), jnp.int32),
            pltpu.VMEM((C2,), jnp.int32),
            pltpu.VMEM((C2,), jnp.int32),
            pltpu.VMEM((C2,), jnp.int32),
            pltpu.VMEM((C2,), jnp.int32),
            pltpu.VMEM((C2,), jnp.int32),
            pltpu.VMEM_SHARED((ACC_ROWS, 64), jnp.float32),
            pltpu.SemaphoreType.DMA,
            pltpu.SemaphoreType.DMA,
            pltpu.SemaphoreType.DMA,
            pltpu.SemaphoreType.DMA,
        ],
    )(s2_tab, r2_tab, alpha_e, sub, rel, obj)

    hidden_new, path_out = pl.pallas_call(
        _final_body,
        out_shape=(
            jax.ShapeDtypeStruct((n_node, 128), jnp.float32),
            jax.ShapeDtypeStruct((n_node, 64), jnp.float32),
        ),
        grid=(10,),
        in_specs=[
            pl.BlockSpec((N_CORES, blk, 128), lambda i: (0, i, 0)),
            pl.BlockSpec((N_CORES, blk, 64), lambda i: (0, i, 0)),
            pl.BlockSpec((128, 128), lambda i: (0, 0)),
            pl.BlockSpec(memory_space=pltpu.SMEM),
        ],
        out_specs=(
            pl.BlockSpec((blk, 128), lambda i: (i, 0)),
            pl.BlockSpec((blk, 64), lambda i: (i, 0)),
        ),
    )(msg_part, path_part, W_h, c_arr)
    return (hidden_new, path_out)


# speedup vs baseline: 1.0608x; 1.0608x over previous
import jax
import jax.numpy as jnp
from jax import lax
from jax.experimental import pallas as pl
from jax.experimental.pallas import tpu as pltpu
from jax.experimental.pallas import tpu_sc as plsc

MIN_NORM = 1e-15
BALL_EPS = 0.004
MIN_CURVATURE = 1e-06
LN2 = 0.6931471805599453

N_SUBCORES = 16
N_CORES = 2
E_PER_TILE = 10080
ACC_ROWS = 10008
S1W = 272
S2W = 64
C1 = 24
C2 = 80


def _expmap0_tc(u, c):
    sqrt_c = jnp.sqrt(c)
    u_norm = jnp.maximum(
        jnp.sqrt(jnp.sum(u * u, axis=-1, keepdims=True)), MIN_NORM)
    g = jnp.tanh(jnp.clip(sqrt_c * u_norm, -15.0, 15.0)) * u / (sqrt_c * u_norm)
    maxnorm = (1.0 - BALL_EPS) / sqrt_c
    g_norm = jnp.maximum(
        jnp.sqrt(jnp.sum(g * g, axis=-1, keepdims=True)), MIN_NORM)
    return jnp.where(g_norm > maxnorm, g / g_norm * maxnorm, g)


def _stable_body(h_ref, p_ref, hq_ref, ws_ref, wpp_ref, wq_ref, wqb_ref, c_ref,
                 s1_ref, s2_ref, q_ref):
    c = c_ref[0]
    h = h_ref[...]
    a = jnp.dot(h, ws_ref[...], preferred_element_type=jnp.float32)
    hh = _expmap0_tc(h, c)
    x2 = jnp.sum(hh * hh, axis=-1, keepdims=True)
    p1 = jnp.dot(p_ref[...], wpp_ref[...], preferred_element_type=jnp.float32)
    pad = jnp.zeros((h.shape[0], S1W - 257), jnp.float32)
    s1_ref[...] = jnp.concatenate([a, hh, x2, pad], axis=1)
    s2_ref[...] = p1
    q_ref[...] = (jnp.dot(hq_ref[...], wq_ref[...],
                          preferred_element_type=jnp.float32) + wqb_ref[...])


def _rtable_body(r_ref, wr_ref, wpr_ref, c_ref, r1_ref, r2_ref):
    c = c_ref[0]
    r = r_ref[...]
    b = jnp.dot(r, wr_ref[...], preferred_element_type=jnp.float32)
    rh = _expmap0_tc(r, c)
    y2 = jnp.sum(rh * rh, axis=-1, keepdims=True)
    p2 = jnp.dot(r, wpr_ref[...], preferred_element_type=jnp.float32)
    pad = jnp.zeros((r.shape[0], S1W - 257), jnp.float32)
    r1_ref[...] = jnp.concatenate([b, rh, y2, pad], axis=1)
    r2_ref[...] = p2


def _final_body(pm_ref, pp_ref, wh_ref, c_ref, hid_ref, path_ref):
    c = c_ref[0]
    sqrt_c = jnp.sqrt(c)
    m = pm_ref[0] + pm_ref[1]
    a = jnp.dot(m, wh_ref[...], preferred_element_type=jnp.float32)
    a_norm = jnp.maximum(
        jnp.sqrt(jnp.sum(a * a, axis=-1, keepdims=True)), MIN_NORM)
    g = jnp.tanh(jnp.clip(sqrt_c * a_norm, -15.0, 15.0)) * a / (sqrt_c * a_norm)
    maxnorm = (1.0 - BALL_EPS) / sqrt_c
    g_norm = jnp.maximum(
        jnp.sqrt(jnp.sum(g * g, axis=-1, keepdims=True)), MIN_NORM)
    y = jnp.where(g_norm > maxnorm, g / g_norm * maxnorm, g)
    y_norm = jnp.maximum(
        jnp.sqrt(jnp.sum(y * y, axis=-1, keepdims=True)), MIN_NORM)
    z = jnp.clip(sqrt_c * y_norm, -1.0 + 1e-05, 1.0 - 1e-05)
    art = 0.5 * (jnp.log1p(z) - jnp.log1p(-z))
    hid_ref[...] = y / y_norm / sqrt_c * art
    path_ref[...] = pp_ref[0] + pp_ref[1]


def _lane_sum(v):
    iota = lax.broadcasted_iota(jnp.int32, (16,), 0)
    for k in (8, 4, 2, 1):
        v = v + v.at[jnp.bitwise_xor(iota, k)].get(
            mode="promise_in_bounds")
    return v


def _lane0(v):
    return v.at[jnp.zeros((16,), jnp.int32)].get(mode="promise_in_bounds")


def _attention_alpha(sbuf, rbuf, qbuf, i, wv, wb_vec):
    aacc = None
    for j in range(8):
        av = sbuf[i, pl.ds(j * 16, 16)]
        bv = rbuf[i, pl.ds(j * 16, 16)]
        qv = qbuf[i, pl.ds(j * 16, 16)]
        t = jnp.maximum(av + bv + qv, 0.0) * wv[j]
        aacc = t if aacc is None else aacc + t
    lo_v = jnp.full((16,), jnp.sum(aacc), jnp.float32) + wb_vec
    return 1.0 / (1.0 + jnp.exp(-lo_v))


ZCH = 24


def _zero_and_barrier(stage, acc, sid, stage_rows, width):
    zeros16 = jnp.zeros((16,), jnp.float32)

    def zrow(i, carry):
        for j in range(width // 16):
            stage[i, pl.ds(j * 16, 16)] = zeros16
        return carry
    lax.fori_loop(0, min(stage_rows, ZCH), zrow, 0)
    src = stage if stage_rows == ZCH else stage.at[pl.ds(0, ZCH)]
    n_zchunk = ACC_ROWS // ZCH

    def zacc(k, carry):
        cidx = sid + k * N_SUBCORES

        @pl.when(cidx < n_zchunk)
        def _():
            off = pl.multiple_of(cidx * ZCH, 8)
            pltpu.sync_copy(src, acc.at[pl.ds(off, ZCH)])
        return carry
    lax.fori_loop(0, (n_zchunk + N_SUBCORES - 1) // N_SUBCORES, zacc, 0)
    plsc.subcore_barrier()


def _writeout(acc, out_hbm, cid, sid, n_rows, chunk):
    n_zchunk = n_rows // chunk

    def wout(k, carry):
        cidx = sid + k * N_SUBCORES

        @pl.when(cidx < n_zchunk)
        def _():
            off = pl.multiple_of(cidx * chunk, 8)
            pltpu.sync_copy(acc.at[pl.ds(off, chunk)],
                            out_hbm.at[cid, pl.ds(off, chunk)])
        return carry
    lax.fori_loop(0, (n_zchunk + N_SUBCORES - 1) // N_SUBCORES, wout, 0)


def _msg_body(s_hbm, r_hbm, q_hbm, sub_hbm, rel_hbm, ridx_hbm, obj_hbm,
              w_hbm, cst_hbm, out_hbm, aout_hbm,
              sbuf0, rbuf0, qbuf0, sbuf1, rbuf1, qbuf1, stage, abuf,
              isub0, irel0, iridx0, iobj0, isub1, irel1, iridx1, iobj1,
              wbuf, cbuf, acc, semi0, semi1, semg0, semg1):
    cid = lax.axis_index("c")
    sid = lax.axis_index("s")
    tid = cid * N_SUBCORES + sid

    pltpu.sync_copy(w_hbm, wbuf)
    pltpu.sync_copy(cst_hbm, cbuf)
    wv = [wbuf[j, :] for j in range(8)]
    c_vec = cbuf[0, :]
    sqrtc_vec = cbuf[1, :]
    invsqrtc_vec = cbuf[2, :]
    maxnorm_vec = cbuf[3, :]
    wb_vec = cbuf[4, :]
    c2_vec = cbuf[5, :]
    one_i = jnp.full((16,), 1, jnp.int32)
    k23_i = jnp.full((16,), 23, jnp.int32)
    lane0 = lax.broadcasted_iota(jnp.int32, (16,), 0) == 0

    _zero_and_barrier(stage, acc, sid, C1, 128)

    idxs = [(isub0, irel0, iridx0, iobj0), (isub1, irel1, iridx1, iobj1)]
    bufs = [(sbuf0, rbuf0, qbuf0), (sbuf1, rbuf1, qbuf1)]
    semi = [semi0, semi1]
    semg = [semg0, semg1]

    def fire_idx(t, b):
        base = pl.multiple_of(tid * E_PER_TILE + t * C1, 8)
        for src, dst in zip((sub_hbm, rel_hbm, ridx_hbm, obj_hbm), idxs[b]):
            pltpu.async_copy(src.at[pl.ds(base, C1)], dst, semi[b])

    def wait_idx(b):
        for src, dst in zip((sub_hbm, rel_hbm, ridx_hbm, obj_hbm), idxs[b]):
            pltpu.make_async_copy(src.at[pl.ds(0, C1)], dst, semi[b]).wait()

    def fire_gathers(b):
        isub, irel, iridx, _ = idxs[b]
        sb, rb, qb = bufs[b]
        pltpu.async_copy(s_hbm.at[isub], sb, semg[b])
        pltpu.async_copy(r_hbm.at[irel], rb, semg[b])
        pltpu.async_copy(q_hbm.at[iridx], qb, semg[b])

    def wait_gathers(b):
        isub, irel, iridx, _ = idxs[b]
        sb, rb, qb = bufs[b]
        pltpu.make_async_copy(s_hbm.at[isub], sb, semg[b]).wait()
        pltpu.make_async_copy(r_hbm.at[irel], rb, semg[b]).wait()
        pltpu.make_async_copy(q_hbm.at[iridx], qb, semg[b]).wait()

    def do_chunk(t, b):
        sb, rb, qb = bufs[b]
        iobj = idxs[b][3]

        @plsc.parallel_loop(0, C1, unroll=4)
        def edge_one(i):
            alpha = _attention_alpha(sb, rb, qb, i, wv, wb_vec)
            plsc.store_scatter(abuf, [jnp.full((16,), i, jnp.int32)], alpha,
                               mask=lane0)
            xs = []
            ys = []
            xyacc = None
            for j in range(8):
                xv = sb[i, pl.ds(128 + j * 16, 16)]
                yv = rb[i, pl.ds(128 + j * 16, 16)]
                xs.append(xv)
                ys.append(yv)
                pv = xv * yv
                xyacc = pv if xyacc is None else xyacc + pv
            xy = jnp.full((16,), jnp.sum(xyacc), jnp.float32)
            x2 = _lane0(sb[i, pl.ds(256, 16)])
            y2 = _lane0(rb[i, pl.ds(256, 16)])

            two_cxy = 2.0 * (c_vec * xy)
            den = jnp.maximum(1.0 + two_cxy + c2_vec * (x2 * y2), MIN_NORM)
            u = (1.0 + two_cxy + c_vec * y2) / den
            v = (1.0 - c_vec * x2) / den
            n2 = u * u * x2 + 2.0 * (u * v * xy) + v * v * y2
            n2g = jnp.maximum(n2, 1e-30)
            bits = plsc.bitcast(n2g, jnp.int32)
            k0 = (jnp.full((16,), 0x5F3759DF, jnp.int32) -
                  lax.shift_right_logical(bits, one_i))
            yr = plsc.bitcast(k0, jnp.float32)
            half = 0.5 * n2g
            for _ in range(3):
                yr = yr * (1.5 - half * yr * yr)
            n = n2g * yr
            norm = jnp.maximum(n, MIN_NORM)
            r = jnp.minimum(1.0, maxnorm_vec / norm)
            y_norm = jnp.maximum(r * n, MIN_NORM)
            z = jnp.clip(sqrtc_vec * y_norm, -1.0 + 1e-05, 1.0 - 1e-05)
            qr = (1.0 + z) / (1.0 - z)
            qb_ = plsc.bitcast(qr, jnp.int32)
            e_i = lax.shift_right_logical(qb_, k23_i) - 127
            m_b = jnp.bitwise_or(jnp.bitwise_and(qb_, 0x007FFFFF), 0x3F800000)
            m = plsc.bitcast(m_b, jnp.float32)
            t_ = (m - 1.0) / (m + 1.0)
            t2 = t_ * t_
            lnm = t_ * (2.0 + t2 * (0.6666666666 + t2 * (0.4 + t2 * (
                0.2857142857 + t2 * 0.2222222222))))
            lnq = e_i.astype(jnp.float32) * LN2 + lnm
            art = 0.5 * lnq
            tt = (r / y_norm) * invsqrtc_vec * art
            o1 = alpha * (tt * u)
            o2 = alpha * (tt * v)
            for j in range(8):
                stage[i, pl.ds(j * 16, 16)] = o1 * xs[j] + o2 * ys[j]

        pltpu.sync_copy(stage, acc.at[iobj], add=True)
        base = pl.multiple_of(tid * E_PER_TILE + t * C1, 8)
        pltpu.sync_copy(abuf, aout_hbm.at[pl.ds(base, C1)])

    nch = E_PER_TILE // C1
    fire_idx(jnp.int32(0), 0)
    wait_idx(0)
    fire_gathers(0)
    fire_idx(jnp.int32(1), 1)

    def pair(t2, carry):
        t = t2 * 2
        wait_gathers(0)
        wait_idx(1)
        fire_gathers(1)
        do_chunk(t, 0)

        @pl.when(t + 2 < nch)
        def _():
            fire_idx(t + 2, 0)
        wait_gathers(1)

        @pl.when(t + 2 < nch)
        def _():
            wait_idx(0)
            fire_gathers(0)
        do_chunk(t + 1, 1)

        @pl.when(t + 2 < nch)
        def _():
            fire_idx(t + 3, 1)
        return carry

    lax.fori_loop(0, nch // 2, pair, 0)
    plsc.subcore_barrier()
    _writeout(acc, out_hbm, cid, sid, 10000, 40)


def _path_body(s_hbm, r_hbm, a_hbm, sub_hbm, rel_hbm, obj_hbm,
               out_hbm,
               sbuf0, rbuf0, abuf0, sbuf1, rbuf1, abuf1, stage,
               isub0, irel0, iobj0, isub1, irel1, iobj1,
               acc, semi0, semi1, semg0, semg1):
    cid = lax.axis_index("c")
    sid = lax.axis_index("s")
    tid = cid * N_SUBCORES + sid

    _zero_and_barrier(stage, acc, sid, C2, 64)

    idxs = [(isub0, irel0, iobj0), (isub1, irel1, iobj1)]
    bufs = [(sbuf0, rbuf0, abuf0), (sbuf1, rbuf1, abuf1)]
    semi = [semi0, semi1]
    semg = [semg0, semg1]

    def fire_idx(t, b):
        base = pl.multiple_of(tid * E_PER_TILE + t * C2, 8)
        for src, dst in zip((sub_hbm, rel_hbm, obj_hbm), idxs[b]):
            pltpu.async_copy(src.at[pl.ds(base, C2)], dst, semi[b])
        pltpu.async_copy(a_hbm.at[pl.ds(base, C2)], bufs[b][2], semi[b])

    def wait_idx(b):
        for src, dst in zip((sub_hbm, rel_hbm, obj_hbm), idxs[b]):
            pltpu.make_async_copy(src.at[pl.ds(0, C2)], dst, semi[b]).wait()
        pltpu.make_async_copy(a_hbm.at[pl.ds(0, C2)], bufs[b][2],
                              semi[b]).wait()

    def fire_gathers(b):
        isub, irel, _ = idxs[b]
        sb, rb, _ = bufs[b]
        pltpu.async_copy(s_hbm.at[isub], sb, semg[b])
        pltpu.async_copy(r_hbm.at[irel], rb, semg[b])

    def wait_gathers(b):
        isub, irel, _ = idxs[b]
        sb, rb, _ = bufs[b]
        pltpu.make_async_copy(s_hbm.at[isub], sb, semg[b]).wait()
        pltpu.make_async_copy(r_hbm.at[irel], rb, semg[b]).wait()

    def do_chunk(t, b):
        sb, rb, ab = bufs[b]
        iobj = idxs[b][2]

        @plsc.parallel_loop(0, C2, unroll=4)
        def edge_one(i):
            alpha = plsc.load_gather(ab, [jnp.full((16,), i, jnp.int32)])
            for j in range(4):
                pp = (sb[i, pl.ds(j * 16, 16)] +
                      rb[i, pl.ds(j * 16, 16)])
                pc = jnp.clip(pp, -15.0, 15.0)
                e2 = jnp.exp(2.0 * pc)
                th = (e2 - 1.0) / (e2 + 1.0)
                stage[i, pl.ds(j * 16, 16)] = alpha * th

        pltpu.sync_copy(stage, acc.at[iobj], add=True)

    nch = E_PER_TILE // C2
    fire_idx(jnp.int32(0), 0)
    wait_idx(0)
    fire_gathers(0)
    fire_idx(jnp.int32(1), 1)

    def pair(t2, carry):
        t = t2 * 2
        wait_gathers(0)
        wait_idx(1)
        fire_gathers(1)
        do_chunk(t, 0)

        @pl.when(t + 2 < nch)
        def _():
            fire_idx(t + 2, 0)
        wait_gathers(1)

        @pl.when(t + 2 < nch)
        def _():
            wait_idx(0)
            fire_gathers(0)
        do_chunk(t + 1, 1)

        @pl.when(t + 2 < nch)
        def _():
            fire_idx(t + 3, 1)
        return carry

    lax.fori_loop(0, nch // 2, pair, 0)
    plsc.subcore_barrier()
    _writeout(acc, out_hbm, cid, sid, 10000, 80)


def kernel(q_sub, q_rel, hidden, path_state, edges, nodes, old_nodes_new_idx,
           batchsize, rela_embed, Ws_attn, Wr_attn, Wqr_attn_w, Wqr_attn_b,
           w_alpha_w, w_alpha_b, W_h, W_path_prev, W_path_rel, curvature):
    c = jnp.maximum(curvature, MIN_CURVATURE)
    n_node = hidden.shape[0]
    vocab = rela_embed.shape[0]

    n_edge = edges.shape[0]
    n_tiles = N_CORES * N_SUBCORES
    epad = E_PER_TILE - n_edge // n_tiles

    def _pad_col(col, fill):
        col = jnp.asarray(col, jnp.int32).reshape(n_tiles, -1)
        col = jnp.pad(col, ((0, 0), (0, epad)), constant_values=fill)
        return col.reshape(-1)

    sub = _pad_col(edges[:, 4], 0)
    rel = _pad_col(edges[:, 2], 0)
    obj = _pad_col(edges[:, 5], n_node)
    r_idx = _pad_col(edges[:, 0], 0)
    vpad = (-vocab) % 16
    rela_p = jnp.pad(rela_embed, ((0, vpad), (0, 0)))
    vp = vocab + vpad
    hq = jnp.take(rela_embed, q_rel, axis=0)
    c_arr = jnp.reshape(c, (1,))
    sqrt_c = jnp.sqrt(c)
    consts = jnp.stack([c, sqrt_c, 1.0 / sqrt_c, (1.0 - BALL_EPS) / sqrt_c,
                        w_alpha_b[0], c * c, jnp.float32(0), jnp.float32(0)])
    consts = jnp.tile(consts[:, None], (1, 16))
    w_r = jnp.reshape(w_alpha_w, (8, 16))
    wqb = jnp.reshape(Wqr_attn_b, (1, 128))

    blk = n_node // 10
    s1_tab, s2_tab, q_tab = pl.pallas_call(
        _stable_body,
        out_shape=(
            jax.ShapeDtypeStruct((n_node, S1W), jnp.float32),
            jax.ShapeDtypeStruct((n_node, S2W), jnp.float32),
            jax.ShapeDtypeStruct((n_node, 128), jnp.float32),
        ),
        grid=(10,),
        in_specs=[
            pl.BlockSpec((blk, 128), lambda i: (i, 0)),
            pl.BlockSpec((blk, 64), lambda i: (i, 0)),
            pl.BlockSpec((blk, 128), lambda i: (i, 0)),
            pl.BlockSpec((128, 128), lambda i: (0, 0)),
            pl.BlockSpec((64, 64), lambda i: (0, 0)),
            pl.BlockSpec((128, 128), lambda i: (0, 0)),
            pl.BlockSpec((1, 128), lambda i: (0, 0)),
            pl.BlockSpec(memory_space=pltpu.SMEM),
        ],
        out_specs=(
            pl.BlockSpec((blk, S1W), lambda i: (i, 0)),
            pl.BlockSpec((blk, S2W), lambda i: (i, 0)),
            pl.BlockSpec((blk, 128), lambda i: (i, 0)),
        ),
    )(hidden, path_state, hq, Ws_attn, W_path_prev, Wqr_attn_w, wqb, c_arr)

    rblk = vp // 2
    r1_tab, r2_tab = pl.pallas_call(
        _rtable_body,
        out_shape=(
            jax.ShapeDtypeStruct((vp, S1W), jnp.float32),
            jax.ShapeDtypeStruct((vp, S2W), jnp.float32),
        ),
        grid=(2,),
        in_specs=[
            pl.BlockSpec((rblk, 128), lambda i: (i, 0)),
            pl.BlockSpec((128, 128), lambda i: (0, 0)),
            pl.BlockSpec((128, 64), lambda i: (0, 0)),
            pl.BlockSpec(memory_space=pltpu.SMEM),
        ],
        out_specs=(
            pl.BlockSpec((rblk, S1W), lambda i: (i, 0)),
            pl.BlockSpec((rblk, S2W), lambda i: (i, 0)),
        ),
    )(rela_p, Wr_attn, W_path_rel, c_arr)

    mesh = plsc.VectorSubcoreMesh(core_axis_name="c", subcore_axis_name="s")
    sc_params = pltpu.CompilerParams(
        needs_layout_passes=False, use_tc_tiling_on_sc=False)

    msg_part, alpha_e = pl.kernel(
        _msg_body,
        out_type=(
            jax.ShapeDtypeStruct((N_CORES, n_node, 128), jnp.float32),
            jax.ShapeDtypeStruct((n_tiles * E_PER_TILE,), jnp.float32),
        ),
        mesh=mesh,
        compiler_params=sc_params,
        scratch_types=[
            pltpu.VMEM((C1, S1W), jnp.float32),
            pltpu.VMEM((C1, S1W), jnp.float32),
            pltpu.VMEM((C1, 128), jnp.float32),
            pltpu.VMEM((C1, S1W), jnp.float32),
            pltpu.VMEM((C1, S1W), jnp.float32),
            pltpu.VMEM((C1, 128), jnp.float32),
            pltpu.VMEM((C1, 128), jnp.float32),
            pltpu.VMEM((C1,), jnp.float32),
            pltpu.VMEM((C1,), jnp.int32),
            pltpu.VMEM((C1,), jnp.int32),
            pltpu.VMEM((C1,), jnp.int32),
            pltpu.VMEM((C1,), jnp.int32),
            pltpu.VMEM((C1,), jnp.int32),
            pltpu.VMEM((C1,), jnp.int32),
            pltpu.VMEM((C1,), jnp.int32),
            pltpu.VMEM((C1,), jnp.int32),
            pltpu.VMEM((8, 16), jnp.float32),
            pltpu.VMEM((8, 16), jnp.float32),
            pltpu.VMEM_SHARED((ACC_ROWS, 128), jnp.float32),
            pltpu.SemaphoreType.DMA,
            pltpu.SemaphoreType.DMA,
            pltpu.SemaphoreType.DMA,
            pltpu.SemaphoreType.DMA,
        ],
    )(s1_tab, r1_tab, q_tab, sub, rel, r_idx, obj, w_r, consts)

    path_part = pl.kernel(
        _path_body,
        out_type=jax.ShapeDtypeStruct((N_CORES, n_node, 64), jnp.float32),
        mesh=mesh,
        compiler_params=sc_params,
        scratch_types=[
            pltpu.VMEM((C2, S2W), jnp.float32),
            pltpu.VMEM((C2, S2W), jnp.float32),
            pltpu.VMEM((C2,), jnp.float32),
            pltpu.VMEM((C2, S2W), jnp.float32),
            pltpu.VMEM((C2, S2W), jnp.float32),
            pltpu.VMEM((C2,), jnp.float32),
            pltpu.VMEM((C2, 64), jnp.float32),
            pltpu.VMEM((C2,), jnp.int32),
            pltpu.VMEM((C2,), jnp.int32),
            pltpu.VMEM((C2,), jnp.int32),
            pltpu.VMEM((C2,), jnp.int32),
            pltpu.VMEM((C2,), jnp.int32),
            pltpu.VMEM((C2,), jnp.int32),
            pltpu.VMEM_SHARED((ACC_ROWS, 64), jnp.float32),
            pltpu.SemaphoreType.DMA,
            pltpu.SemaphoreType.DMA,
            pltpu.SemaphoreType.DMA,
            pltpu.SemaphoreType.DMA,
        ],
    )(s2_tab, r2_tab, alpha_e, sub, rel, obj)

    hidden_new, path_out = pl.pallas_call(
        _final_body,
        out_shape=(
            jax.ShapeDtypeStruct((n_node, 128), jnp.float32),
            jax.ShapeDtypeStruct((n_node, 64), jnp.float32),
        ),
        grid=(10,),
        in_specs=[
            pl.BlockSpec((N_CORES, blk, 128), lambda i: (0, i, 0)),
            pl.BlockSpec((N_CORES, blk, 64), lambda i: (0, i, 0)),
            pl.BlockSpec((128, 128), lambda i: (0, 0)),
            pl.BlockSpec(memory_space=pltpu.SMEM),
        ],
        out_specs=(
            pl.BlockSpec((blk, 128), lambda i: (i, 0)),
            pl.BlockSpec((blk, 64), lambda i: (i, 0)),
        ),
    )(msg_part, path_part, W_h, c_arr)
    return (hidden_new, path_out)


# speedup vs baseline: 1.0771x; 1.0154x over previous
import jax
import jax.numpy as jnp
from jax import lax
from jax.experimental import pallas as pl
from jax.experimental.pallas import tpu as pltpu
from jax.experimental.pallas import tpu_sc as plsc

MIN_NORM = 1e-15
BALL_EPS = 0.004
MIN_CURVATURE = 1e-06
LN2 = 0.6931471805599453

N_SUBCORES = 16
N_CORES = 2
E_PER_TILE = 10080
ACC_ROWS = 10008
S1W = 272
S2W = 64
C1 = 24
C2 = 80


def _expmap0_tc(u, c):
    sqrt_c = jnp.sqrt(c)
    u_norm = jnp.maximum(
        jnp.sqrt(jnp.sum(u * u, axis=-1, keepdims=True)), MIN_NORM)
    g = jnp.tanh(jnp.clip(sqrt_c * u_norm, -15.0, 15.0)) * u / (sqrt_c * u_norm)
    maxnorm = (1.0 - BALL_EPS) / sqrt_c
    g_norm = jnp.maximum(
        jnp.sqrt(jnp.sum(g * g, axis=-1, keepdims=True)), MIN_NORM)
    return jnp.where(g_norm > maxnorm, g / g_norm * maxnorm, g)


def _stable_body(h_ref, p_ref, hq_ref, ws_ref, wpp_ref, wq_ref, wqb_ref, c_ref,
                 s1_ref, s2_ref, q_ref):
    c = c_ref[0]
    h = h_ref[...]
    a = jnp.dot(h, ws_ref[...], preferred_element_type=jnp.float32)
    hh = _expmap0_tc(h, c)
    x2 = jnp.sum(hh * hh, axis=-1, keepdims=True)
    p1 = jnp.dot(p_ref[...], wpp_ref[...], preferred_element_type=jnp.float32)
    pad = jnp.zeros((h.shape[0], S1W - 257), jnp.float32)
    s1_ref[...] = jnp.concatenate([a, hh, x2, pad], axis=1)
    s2_ref[...] = p1
    q_ref[...] = (jnp.dot(hq_ref[...], wq_ref[...],
                          preferred_element_type=jnp.float32) + wqb_ref[...])


def _rtable_body(r_ref, wr_ref, wpr_ref, c_ref, r1_ref, r2_ref):
    c = c_ref[0]
    r = r_ref[...]
    b = jnp.dot(r, wr_ref[...], preferred_element_type=jnp.float32)
    rh = _expmap0_tc(r, c)
    y2 = jnp.sum(rh * rh, axis=-1, keepdims=True)
    p2 = jnp.dot(r, wpr_ref[...], preferred_element_type=jnp.float32)
    pad = jnp.zeros((r.shape[0], S1W - 257), jnp.float32)
    r1_ref[...] = jnp.concatenate([b, rh, y2, pad], axis=1)
    r2_ref[...] = p2


def _final_body(pm_ref, pp_ref, wh_ref, c_ref, hid_ref, path_ref):
    c = c_ref[0]
    sqrt_c = jnp.sqrt(c)
    m = pm_ref[0] + pm_ref[1]
    a = jnp.dot(m, wh_ref[...], preferred_element_type=jnp.float32)
    a_norm = jnp.maximum(
        jnp.sqrt(jnp.sum(a * a, axis=-1, keepdims=True)), MIN_NORM)
    g = jnp.tanh(jnp.clip(sqrt_c * a_norm, -15.0, 15.0)) * a / (sqrt_c * a_norm)
    maxnorm = (1.0 - BALL_EPS) / sqrt_c
    g_norm = jnp.maximum(
        jnp.sqrt(jnp.sum(g * g, axis=-1, keepdims=True)), MIN_NORM)
    y = jnp.where(g_norm > maxnorm, g / g_norm * maxnorm, g)
    y_norm = jnp.maximum(
        jnp.sqrt(jnp.sum(y * y, axis=-1, keepdims=True)), MIN_NORM)
    z = jnp.clip(sqrt_c * y_norm, -1.0 + 1e-05, 1.0 - 1e-05)
    art = 0.5 * (jnp.log1p(z) - jnp.log1p(-z))
    hid_ref[...] = y / y_norm / sqrt_c * art
    path_ref[...] = pp_ref[0] + pp_ref[1]


def _attention_alpha(sbuf, rbuf, qbuf, i, wv, wb_vec):
    aacc = None
    for j in range(8):
        av = sbuf[i, pl.ds(j * 16, 16)]
        bv = rbuf[i, pl.ds(j * 16, 16)]
        qv = qbuf[i, pl.ds(j * 16, 16)]
        t = jnp.maximum(av + bv + qv, 0.0) * wv[j]
        aacc = t if aacc is None else aacc + t
    lo_v = jnp.full((16,), jnp.sum(aacc), jnp.float32) + wb_vec
    return 1.0 / (1.0 + jnp.exp(-lo_v))


ZCH = 24


def _zero_and_barrier(stage, acc, sid, stage_rows, width):
    zeros16 = jnp.zeros((16,), jnp.float32)

    def zrow(i, carry):
        for j in range(width // 16):
            stage[i, pl.ds(j * 16, 16)] = zeros16
        return carry
    lax.fori_loop(0, min(stage_rows, ZCH), zrow, 0)
    src = stage if stage_rows == ZCH else stage.at[pl.ds(0, ZCH)]
    n_zchunk = ACC_ROWS // ZCH

    def zacc(k, carry):
        cidx = sid + k * N_SUBCORES

        @pl.when(cidx < n_zchunk)
        def _():
            off = pl.multiple_of(cidx * ZCH, 8)
            pltpu.sync_copy(src, acc.at[pl.ds(off, ZCH)])
        return carry
    lax.fori_loop(0, (n_zchunk + N_SUBCORES - 1) // N_SUBCORES, zacc, 0)
    plsc.subcore_barrier()


def _writeout(acc, out_hbm, cid, sid, n_rows, chunk):
    n_zchunk = n_rows // chunk

    def wout(k, carry):
        cidx = sid + k * N_SUBCORES

        @pl.when(cidx < n_zchunk)
        def _():
            off = pl.multiple_of(cidx * chunk, 8)
            pltpu.sync_copy(acc.at[pl.ds(off, chunk)],
                            out_hbm.at[cid, pl.ds(off, chunk)])
        return carry
    lax.fori_loop(0, (n_zchunk + N_SUBCORES - 1) // N_SUBCORES, wout, 0)


def _msg_body(s_hbm, r_hbm, q_hbm, sub_hbm, rel_hbm, ridx_hbm, obj_hbm,
              w_hbm, cst_hbm, out_hbm, aout_hbm,
              sbuf0, rbuf0, qbuf0, sbuf1, rbuf1, qbuf1, stage, abuf,
              isub0, irel0, iridx0, iobj0, isub1, irel1, iridx1, iobj1,
              wbuf, cbuf, acc, semi0, semi1, semg0, semg1):
    cid = lax.axis_index("c")
    sid = lax.axis_index("s")
    tid = cid * N_SUBCORES + sid

    pltpu.sync_copy(w_hbm, wbuf)
    pltpu.sync_copy(cst_hbm, cbuf)
    wv = [wbuf[j, :] for j in range(8)]
    c_vec = cbuf[0, :]
    sqrtc_vec = cbuf[1, :]
    invsqrtc_vec = cbuf[2, :]
    maxnorm_vec = cbuf[3, :]
    wb_vec = cbuf[4, :]
    c2_vec = cbuf[5, :]
    one_i = jnp.full((16,), 1, jnp.int32)
    k23_i = jnp.full((16,), 23, jnp.int32)
    lane0 = lax.broadcasted_iota(jnp.int32, (16,), 0) == 0

    _zero_and_barrier(stage, acc, sid, C1, 128)

    idxs = [(isub0, irel0, iridx0, iobj0), (isub1, irel1, iridx1, iobj1)]
    bufs = [(sbuf0, rbuf0, qbuf0), (sbuf1, rbuf1, qbuf1)]
    semi = [semi0, semi1]
    semg = [semg0, semg1]

    def fire_idx(t, b):
        base = pl.multiple_of(tid * E_PER_TILE + t * C1, 8)
        for src, dst in zip((sub_hbm, rel_hbm, ridx_hbm, obj_hbm), idxs[b]):
            pltpu.async_copy(src.at[pl.ds(base, C1)], dst, semi[b])

    def wait_idx(b):
        for src, dst in zip((sub_hbm, rel_hbm, ridx_hbm, obj_hbm), idxs[b]):
            pltpu.make_async_copy(src.at[pl.ds(0, C1)], dst, semi[b]).wait()

    def fire_gathers(b):
        isub, irel, iridx, _ = idxs[b]
        sb, rb, qb = bufs[b]
        pltpu.async_copy(s_hbm.at[isub], sb, semg[b])
        pltpu.async_copy(r_hbm.at[irel], rb, semg[b])
        pltpu.async_copy(q_hbm.at[iridx], qb, semg[b])

    def wait_gathers(b):
        isub, irel, iridx, _ = idxs[b]
        sb, rb, qb = bufs[b]
        pltpu.make_async_copy(s_hbm.at[isub], sb, semg[b]).wait()
        pltpu.make_async_copy(r_hbm.at[irel], rb, semg[b]).wait()
        pltpu.make_async_copy(q_hbm.at[iridx], qb, semg[b]).wait()

    def do_chunk(t, b):
        sb, rb, qb = bufs[b]
        iobj = idxs[b][3]

        @plsc.parallel_loop(0, C1, unroll=4)
        def edge_one(i):
            alpha = _attention_alpha(sb, rb, qb, i, wv, wb_vec)
            plsc.store_scatter(abuf, [jnp.full((16,), i, jnp.int32)], alpha,
                               mask=lane0)
            xs = []
            ys = []
            xyacc = None
            for j in range(8):
                xv = sb[i, pl.ds(128 + j * 16, 16)]
                yv = rb[i, pl.ds(128 + j * 16, 16)]
                xs.append(xv)
                ys.append(yv)
                pv = xv * yv
                xyacc = pv if xyacc is None else xyacc + pv
            xy = jnp.full((16,), jnp.sum(xyacc), jnp.float32)
            x2 = jnp.full((16,), jnp.sum(sb[i, pl.ds(256, 16)]), jnp.float32)
            y2 = jnp.full((16,), jnp.sum(rb[i, pl.ds(256, 16)]), jnp.float32)

            two_cxy = 2.0 * (c_vec * xy)
            den = jnp.maximum(1.0 + two_cxy + c2_vec * (x2 * y2), MIN_NORM)
            u = (1.0 + two_cxy + c_vec * y2) / den
            v = (1.0 - c_vec * x2) / den
            n2 = u * u * x2 + 2.0 * (u * v * xy) + v * v * y2
            n2g = jnp.maximum(n2, 1e-30)
            bits = plsc.bitcast(n2g, jnp.int32)
            k0 = (jnp.full((16,), 0x5F3759DF, jnp.int32) -
                  lax.shift_right_logical(bits, one_i))
            yr = plsc.bitcast(k0, jnp.float32)
            half = 0.5 * n2g
            for _ in range(3):
                yr = yr * (1.5 - half * yr * yr)
            n = n2g * yr
            norm = jnp.maximum(n, MIN_NORM)
            r = jnp.minimum(1.0, maxnorm_vec / norm)
            y_norm = jnp.maximum(r * n, MIN_NORM)
            z = jnp.clip(sqrtc_vec * y_norm, -1.0 + 1e-05, 1.0 - 1e-05)
            qr = (1.0 + z) / (1.0 - z)
            qb_ = plsc.bitcast(qr, jnp.int32)
            e_i = lax.shift_right_logical(qb_, k23_i) - 127
            m_b = jnp.bitwise_or(jnp.bitwise_and(qb_, 0x007FFFFF), 0x3F800000)
            m = plsc.bitcast(m_b, jnp.float32)
            t_ = (m - 1.0) / (m + 1.0)
            t2 = t_ * t_
            lnm = t_ * (2.0 + t2 * (0.6666666666 + t2 * (0.4 + t2 * (
                0.2857142857 + t2 * 0.2222222222))))
            lnq = e_i.astype(jnp.float32) * LN2 + lnm
            art = 0.5 * lnq
            tt = (r / y_norm) * invsqrtc_vec * art
            o1 = alpha * (tt * u)
            o2 = alpha * (tt * v)
            for j in range(8):
                stage[i, pl.ds(j * 16, 16)] = o1 * xs[j] + o2 * ys[j]

        pltpu.sync_copy(stage, acc.at[iobj], add=True)
        base = pl.multiple_of(tid * E_PER_TILE + t * C1, 8)
        pltpu.sync_copy(abuf, aout_hbm.at[pl.ds(base, C1)])

    nch = E_PER_TILE // C1
    fire_idx(jnp.int32(0), 0)
    wait_idx(0)
    fire_gathers(0)
    fire_idx(jnp.int32(1), 1)

    def pair(t2, carry):
        t = t2 * 2
        wait_gathers(0)
        wait_idx(1)
        fire_gathers(1)
        do_chunk(t, 0)

        @pl.when(t + 2 < nch)
        def _():
            fire_idx(t + 2, 0)
        wait_gathers(1)

        @pl.when(t + 2 < nch)
        def _():
            wait_idx(0)
            fire_gathers(0)
        do_chunk(t + 1, 1)

        @pl.when(t + 2 < nch)
        def _():
            fire_idx(t + 3, 1)
        return carry

    lax.fori_loop(0, nch // 2, pair, 0)
    plsc.subcore_barrier()
    _writeout(acc, out_hbm, cid, sid, 10000, 40)


def _path_body(s_hbm, r_hbm, a_hbm, sub_hbm, rel_hbm, obj_hbm,
               out_hbm,
               sbuf0, rbuf0, abuf0, sbuf1, rbuf1, abuf1, stage,
               isub0, irel0, iobj0, isub1, irel1, iobj1,
               acc, semi0, semi1, semg0, semg1):
    cid = lax.axis_index("c")
    sid = lax.axis_index("s")
    tid = cid * N_SUBCORES + sid

    _zero_and_barrier(stage, acc, sid, C2, 64)

    idxs = [(isub0, irel0, iobj0), (isub1, irel1, iobj1)]
    bufs = [(sbuf0, rbuf0, abuf0), (sbuf1, rbuf1, abuf1)]
    semi = [semi0, semi1]
    semg = [semg0, semg1]

    def fire_idx(t, b):
        base = pl.multiple_of(tid * E_PER_TILE + t * C2, 8)
        for src, dst in zip((sub_hbm, rel_hbm, obj_hbm), idxs[b]):
            pltpu.async_copy(src.at[pl.ds(base, C2)], dst, semi[b])
        pltpu.async_copy(a_hbm.at[pl.ds(base, C2)], bufs[b][2], semi[b])

    def wait_idx(b):
        for src, dst in zip((sub_hbm, rel_hbm, obj_hbm), idxs[b]):
            pltpu.make_async_copy(src.at[pl.ds(0, C2)], dst, semi[b]).wait()
        pltpu.make_async_copy(a_hbm.at[pl.ds(0, C2)], bufs[b][2],
                              semi[b]).wait()

    def fire_gathers(b):
        isub, irel, _ = idxs[b]
        sb, rb, _ = bufs[b]
        pltpu.async_copy(s_hbm.at[isub], sb, semg[b])
        pltpu.async_copy(r_hbm.at[irel], rb, semg[b])

    def wait_gathers(b):
        isub, irel, _ = idxs[b]
        sb, rb, _ = bufs[b]
        pltpu.make_async_copy(s_hbm.at[isub], sb, semg[b]).wait()
        pltpu.make_async_copy(r_hbm.at[irel], rb, semg[b]).wait()

    def do_chunk(t, b):
        sb, rb, ab = bufs[b]
        iobj = idxs[b][2]

        @plsc.parallel_loop(0, C2, unroll=4)
        def edge_one(i):
            alpha = plsc.load_gather(ab, [jnp.full((16,), i, jnp.int32)])
            for j in range(4):
                pp = (sb[i, pl.ds(j * 16, 16)] +
                      rb[i, pl.ds(j * 16, 16)])
                pc = jnp.clip(pp, -15.0, 15.0)
                e2 = jnp.exp(2.0 * pc)
                th = (e2 - 1.0) / (e2 + 1.0)
                stage[i, pl.ds(j * 16, 16)] = alpha * th

        pltpu.sync_copy(stage, acc.at[iobj], add=True)

    nch = E_PER_TILE // C2
    fire_idx(jnp.int32(0), 0)
    wait_idx(0)
    fire_gathers(0)
    fire_idx(jnp.int32(1), 1)

    def pair(t2, carry):
        t = t2 * 2
        wait_gathers(0)
        wait_idx(1)
        fire_gathers(1)
        do_chunk(t, 0)

        @pl.when(t + 2 < nch)
        def _():
            fire_idx(t + 2, 0)
        wait_gathers(1)

        @pl.when(t + 2 < nch)
        def _():
            wait_idx(0)
            fire_gathers(0)
        do_chunk(t + 1, 1)

        @pl.when(t + 2 < nch)
        def _():
            fire_idx(t + 3, 1)
        return carry

    lax.fori_loop(0, nch // 2, pair, 0)
    plsc.subcore_barrier()
    _writeout(acc, out_hbm, cid, sid, 10000, 80)


def kernel(q_sub, q_rel, hidden, path_state, edges, nodes, old_nodes_new_idx,
           batchsize, rela_embed, Ws_attn, Wr_attn, Wqr_attn_w, Wqr_attn_b,
           w_alpha_w, w_alpha_b, W_h, W_path_prev, W_path_rel, curvature):
    c = jnp.maximum(curvature, MIN_CURVATURE)
    n_node = hidden.shape[0]
    vocab = rela_embed.shape[0]

    n_edge = edges.shape[0]
    n_tiles = N_CORES * N_SUBCORES
    epad = E_PER_TILE - n_edge // n_tiles

    def _pad_col(col, fill):
        col = jnp.asarray(col, jnp.int32).reshape(n_tiles, -1)
        col = jnp.pad(col, ((0, 0), (0, epad)), constant_values=fill)
        return col.reshape(-1)

    sub = _pad_col(edges[:, 4], 0)
    rel = _pad_col(edges[:, 2], 0)
    obj = _pad_col(edges[:, 5], n_node)
    r_idx = _pad_col(edges[:, 0], 0)
    vpad = (-vocab) % 16
    rela_p = jnp.pad(rela_embed, ((0, vpad), (0, 0)))
    vp = vocab + vpad
    hq = jnp.take(rela_embed, q_rel, axis=0)
    c_arr = jnp.reshape(c, (1,))
    sqrt_c = jnp.sqrt(c)
    consts = jnp.stack([c, sqrt_c, 1.0 / sqrt_c, (1.0 - BALL_EPS) / sqrt_c,
                        w_alpha_b[0], c * c, jnp.float32(0), jnp.float32(0)])
    consts = jnp.tile(consts[:, None], (1, 16))
    w_r = jnp.reshape(w_alpha_w, (8, 16))
    wqb = jnp.reshape(Wqr_attn_b, (1, 128))

    blk = n_node // 10
    s1_tab, s2_tab, q_tab = pl.pallas_call(
        _stable_body,
        out_shape=(
            jax.ShapeDtypeStruct((n_node, S1W), jnp.float32),
            jax.ShapeDtypeStruct((n_node, S2W), jnp.float32),
            jax.ShapeDtypeStruct((n_node, 128), jnp.float32),
        ),
        grid=(10,),
        in_specs=[
            pl.BlockSpec((blk, 128), lambda i: (i, 0)),
            pl.BlockSpec((blk, 64), lambda i: (i, 0)),
            pl.BlockSpec((blk, 128), lambda i: (i, 0)),
            pl.BlockSpec((128, 128), lambda i: (0, 0)),
            pl.BlockSpec((64, 64), lambda i: (0, 0)),
            pl.BlockSpec((128, 128), lambda i: (0, 0)),
            pl.BlockSpec((1, 128), lambda i: (0, 0)),
            pl.BlockSpec(memory_space=pltpu.SMEM),
        ],
        out_specs=(
            pl.BlockSpec((blk, S1W), lambda i: (i, 0)),
            pl.BlockSpec((blk, S2W), lambda i: (i, 0)),
            pl.BlockSpec((blk, 128), lambda i: (i, 0)),
        ),
    )(hidden, path_state, hq, Ws_attn, W_path_prev, Wqr_attn_w, wqb, c_arr)

    rblk = vp // 2
    r1_tab, r2_tab = pl.pallas_call(
        _rtable_body,
        out_shape=(
            jax.ShapeDtypeStruct((vp, S1W), jnp.float32),
            jax.ShapeDtypeStruct((vp, S2W), jnp.float32),
        ),
        grid=(2,),
        in_specs=[
            pl.BlockSpec((rblk, 128), lambda i: (i, 0)),
            pl.BlockSpec((128, 128), lambda i: (0, 0)),
            pl.BlockSpec((128, 64), lambda i: (0, 0)),
            pl.BlockSpec(memory_space=pltpu.SMEM),
        ],
        out_specs=(
            pl.BlockSpec((rblk, S1W), lambda i: (i, 0)),
            pl.BlockSpec((rblk, S2W), lambda i: (i, 0)),
        ),
    )(rela_p, Wr_attn, W_path_rel, c_arr)

    mesh = plsc.VectorSubcoreMesh(core_axis_name="c", subcore_axis_name="s")
    sc_params = pltpu.CompilerParams(
        needs_layout_passes=False, use_tc_tiling_on_sc=False)

    msg_part, alpha_e = pl.kernel(
        _msg_body,
        out_type=(
            jax.ShapeDtypeStruct((N_CORES, n_node, 128), jnp.float32),
            jax.ShapeDtypeStruct((n_tiles * E_PER_TILE,), jnp.float32),
        ),
        mesh=mesh,
        compiler_params=sc_params,
        scratch_types=[
            pltpu.VMEM((C1, S1W), jnp.float32),
            pltpu.VMEM((C1, S1W), jnp.float32),
            pltpu.VMEM((C1, 128), jnp.float32),
            pltpu.VMEM((C1, S1W), jnp.float32),
            pltpu.VMEM((C1, S1W), jnp.float32),
            pltpu.VMEM((C1, 128), jnp.float32),
            pltpu.VMEM((C1, 128), jnp.float32),
            pltpu.VMEM((C1,), jnp.float32),
            pltpu.VMEM((C1,), jnp.int32),
            pltpu.VMEM((C1,), jnp.int32),
            pltpu.VMEM((C1,), jnp.int32),
            pltpu.VMEM((C1,), jnp.int32),
            pltpu.VMEM((C1,), jnp.int32),
            pltpu.VMEM((C1,), jnp.int32),
            pltpu.VMEM((C1,), jnp.int32),
            pltpu.VMEM((C1,), jnp.int32),
            pltpu.VMEM((8, 16), jnp.float32),
            pltpu.VMEM((8, 16), jnp.float32),
            pltpu.VMEM_SHARED((ACC_ROWS, 128), jnp.float32),
            pltpu.SemaphoreType.DMA,
            pltpu.SemaphoreType.DMA,
            pltpu.SemaphoreType.DMA,
            pltpu.SemaphoreType.DMA,
        ],
    )(s1_tab, r1_tab, q_tab, sub, rel, r_idx, obj, w_r, consts)

    path_part = pl.kernel(
        _path_body,
        out_type=jax.ShapeDtypeStruct((N_CORES, n_node, 64), jnp.float32),
        mesh=mesh,
        compiler_params=sc_params,
        scratch_types=[
            pltpu.VMEM((C2, S2W), jnp.float32),
            pltpu.VMEM((C2, S2W), jnp.float32),
            pltpu.VMEM((C2,), jnp.float32),
            pltpu.VMEM((C2, S2W), jnp.float32),
            pltpu.VMEM((C2, S2W), jnp.float32),
            pltpu.VMEM((C2,), jnp.float32),
            pltpu.VMEM((C2, 64), jnp.float32),
            pltpu.VMEM((C2,), jnp.int32),
            pltpu.VMEM((C2,), jnp.int32),
            pltpu.VMEM((C2,), jnp.int32),
            pltpu.VMEM((C2,), jnp.int32),
            pltpu.VMEM((C2,), jnp.int32),
            pltpu.VMEM((C2,), jnp.int32),
            pltpu.VMEM_SHARED((ACC_ROWS, 64), jnp.float32),
            pltpu.SemaphoreType.DMA,
            pltpu.SemaphoreType.DMA,
            pltpu.SemaphoreType.DMA,
            pltpu.SemaphoreType.DMA,
        ],
    )(s2_tab, r2_tab, alpha_e, sub, rel, obj)

    hidden_new, path_out = pl.pallas_call(
        _final_body,
        out_shape=(
            jax.ShapeDtypeStruct((n_node, 128), jnp.float32),
            jax.ShapeDtypeStruct((n_node, 64), jnp.float32),
        ),
        grid=(10,),
        in_specs=[
            pl.BlockSpec((N_CORES, blk, 128), lambda i: (0, i, 0)),
            pl.BlockSpec((N_CORES, blk, 64), lambda i: (0, i, 0)),
            pl.BlockSpec((128, 128), lambda i: (0, 0)),
            pl.BlockSpec(memory_space=pltpu.SMEM),
        ],
        out_specs=(
            pl.BlockSpec((blk, 128), lambda i: (i, 0)),
            pl.BlockSpec((blk, 64), lambda i: (i, 0)),
        ),
    )(msg_part, path_part, W_h, c_arr)
    return (hidden_new, path_out)
